# Initial kernel scaffold; baseline (speedup 1.0000x reference)
import functools

import jax
import jax.numpy as jnp
from jax import lax
from jax.experimental import pallas as pl
from jax.experimental.pallas import tpu as pltpu
from jax.experimental.pallas import tpu_sc as plsc

B, S, H = 16, 512, 1024
NUM_TAGS = 64
INTER = 4096
NUM_LABELS = 17
TAGGING_RATE = 0.5
EPS = 1e-12

N = B * S

_SC_INFO = plsc.get_sparse_core_info()
_NC = _SC_INFO.num_cores
_NS = _SC_INFO.num_subcores
_NW = _NC * _NS
_PER_W = N // _NW
_CH = 64
_NCH = _PER_W // _CH


def _sc_gather_body(tab_hbm, ids_hbm, out_hbm, idx_v, rows_v, sem):
    wid = lax.axis_index("s") * _NC + lax.axis_index("c")
    base = wid * _PER_W
    for ch in range(_NCH):
        off = base + ch * _CH
        pltpu.sync_copy(ids_hbm.at[pl.ds(off, _CH)], idx_v)
        pltpu.async_copy(tab_hbm.at[idx_v], rows_v, sem).wait()
        pltpu.sync_copy(rows_v, out_hbm.at[pl.ds(off, _CH)])


_sc_gather = pl.kernel(
    _sc_gather_body,
    out_type=jax.ShapeDtypeStruct((N, H), jnp.float32),
    mesh=plsc.VectorSubcoreMesh(core_axis_name="c", subcore_axis_name="s"),
    scratch_types=[
        pltpu.VMEM((_CH,), jnp.int32),
        pltpu.VMEM((_CH, H), jnp.float32),
        pltpu.SemaphoreType.DMA,
    ],
)

_TB = 512


def _tc_body(word, tagged, g1, beta1, W1, b1, W2, b2, g2, beta2, Wout, bout,
             out):
    x = word[...] + TAGGING_RATE * tagged[...]
    mu = jnp.mean(x, axis=-1, keepdims=True)
    var = jnp.mean((x - mu) ** 2, axis=-1, keepdims=True)
    xn = (x - mu) * lax.rsqrt(var + EPS) * g1[...] + beta1[...]
    h = jnp.dot(xn, W1[...], preferred_element_type=jnp.float32) + b1[...]
    h = jnp.maximum(h, 0.0)
    y = jnp.dot(h, W2[...], preferred_element_type=jnp.float32) + b2[...] + xn
    mu2 = jnp.mean(y, axis=-1, keepdims=True)
    var2 = jnp.mean((y - mu2) ** 2, axis=-1, keepdims=True)
    yn = (y - mu2) * lax.rsqrt(var2 + EPS) * g2[...] + beta2[...]
    out[...] = jnp.dot(yn, Wout[...], preferred_element_type=jnp.float32) + bout[...]


def _tc_call(word2d, tagged2d, g1, beta1, W1, b1, W2, b2, g2, beta2, Wout, bout):
    nb = N // _TB
    tok = lambda i: (i, 0)
    const = lambda i: (0, 0)
    vec = pl.BlockSpec((1, H), const)
    return pl.pallas_call(
        _tc_body,
        grid=(nb,),
        in_specs=[
            pl.BlockSpec((_TB, H), tok),
            pl.BlockSpec((_TB, H), tok),
            vec, vec,
            pl.BlockSpec((H, INTER), const),
            pl.BlockSpec((1, INTER), const),
            pl.BlockSpec((INTER, H), const),
            pl.BlockSpec((1, H), const),
            vec, vec,
            pl.BlockSpec((H, NUM_LABELS), const),
            pl.BlockSpec((1, NUM_LABELS), const),
        ],
        out_specs=pl.BlockSpec((_TB, NUM_LABELS), tok),
        out_shape=jax.ShapeDtypeStruct((N, NUM_LABELS), jnp.float32),
        compiler_params=pltpu.CompilerParams(
            dimension_semantics=("arbitrary",),
        ),
    )(word2d, tagged2d, g1, beta1, W1, b1, W2, b2, g2, beta2, Wout, bout)


def kernel(word_embedding, tag_to_spans, tag_embedding, att_gamma, att_beta,
           W1, b1, W2, b2, ff_gamma, ff_beta, Wout, bout):
    ids = tag_to_spans.reshape(N)
    tagged = _sc_gather(tag_embedding, ids)
    out = _tc_call(
        word_embedding.reshape(N, H),
        tagged,
        att_gamma.reshape(1, H), att_beta.reshape(1, H),
        W1, b1.reshape(1, INTER),
        W2, b2.reshape(1, H),
        ff_gamma.reshape(1, H), ff_beta.reshape(1, H),
        Wout, bout.reshape(1, NUM_LABELS),
    )
    return out.reshape(B, S, NUM_LABELS)


# speedup vs baseline: 1.1511x; 1.1511x over previous
import functools

import jax
import jax.numpy as jnp
from jax import lax
from jax.experimental import pallas as pl
from jax.experimental.pallas import tpu as pltpu
from jax.experimental.pallas import tpu_sc as plsc

B, S, H = 16, 512, 1024
NUM_TAGS = 64
INTER = 4096
NUM_LABELS = 17
TAGGING_RATE = 0.5
EPS = 1e-12

N = B * S

_NC = 2
_NS = 16
_NW = _NC * _NS
_PER_W = N // _NW
_CH = 64
_NCH = _PER_W // _CH


def _sc_gather_body(tab_hbm, ids_hbm, out_hbm, idx_v, rows_v, sem):
    wid = lax.axis_index("s") * _NC + lax.axis_index("c")
    base = wid * _PER_W
    for ch in range(_NCH):
        off = base + ch * _CH
        pltpu.sync_copy(ids_hbm.at[pl.ds(off, _CH)], idx_v)
        pltpu.async_copy(tab_hbm.at[idx_v], rows_v, sem).wait()
        pltpu.sync_copy(rows_v, out_hbm.at[pl.ds(off, _CH)])


@functools.cache
def _sc_gather():
    return pl.kernel(
        _sc_gather_body,
        out_type=jax.ShapeDtypeStruct((N, H), jnp.float32),
        mesh=plsc.VectorSubcoreMesh(core_axis_name="c", subcore_axis_name="s"),
        scratch_types=[
            pltpu.VMEM((_CH,), jnp.int32),
            pltpu.VMEM((_CH, H), jnp.float32),
            pltpu.SemaphoreType.DMA,
        ],
    )

_TB = 512


def _tc_body(word, tagged, g1, beta1, W1, b1, W2, b2, g2, beta2, Wout, bout,
             out):
    x = word[...] + TAGGING_RATE * tagged[...]
    mu = jnp.mean(x, axis=-1, keepdims=True)
    var = jnp.mean((x - mu) ** 2, axis=-1, keepdims=True)
    xn = (x - mu) * lax.rsqrt(var + EPS) * g1[...] + beta1[...]
    h = jnp.dot(xn, W1[...], preferred_element_type=jnp.float32) + b1[...]
    h = jnp.maximum(h, 0.0)
    y = jnp.dot(h, W2[...], preferred_element_type=jnp.float32) + b2[...] + xn
    mu2 = jnp.mean(y, axis=-1, keepdims=True)
    var2 = jnp.mean((y - mu2) ** 2, axis=-1, keepdims=True)
    yn = (y - mu2) * lax.rsqrt(var2 + EPS) * g2[...] + beta2[...]
    out[...] = jnp.dot(yn, Wout[...], preferred_element_type=jnp.float32) + bout[...]


def _tc_call(word2d, tagged2d, g1, beta1, W1, b1, W2, b2, g2, beta2, Wout, bout):
    nb = N // _TB
    tok = lambda i: (i, 0)
    const = lambda i: (0, 0)
    vec = pl.BlockSpec((1, H), const)
    return pl.pallas_call(
        _tc_body,
        grid=(nb,),
        in_specs=[
            pl.BlockSpec((_TB, H), tok),
            pl.BlockSpec((_TB, H), tok),
            vec, vec,
            pl.BlockSpec((H, INTER), const),
            pl.BlockSpec((1, INTER), const),
            pl.BlockSpec((INTER, H), const),
            pl.BlockSpec((1, H), const),
            vec, vec,
            pl.BlockSpec((H, NUM_LABELS), const),
            pl.BlockSpec((1, NUM_LABELS), const),
        ],
        out_specs=pl.BlockSpec((_TB, NUM_LABELS), tok),
        out_shape=jax.ShapeDtypeStruct((N, NUM_LABELS), jnp.float32),
        compiler_params=pltpu.CompilerParams(
            dimension_semantics=("arbitrary",),
        ),
    )(word2d, tagged2d, g1, beta1, W1, b1, W2, b2, g2, beta2, Wout, bout)


def kernel(word_embedding, tag_to_spans, tag_embedding, att_gamma, att_beta,
           W1, b1, W2, b2, ff_gamma, ff_beta, Wout, bout):
    ids = tag_to_spans.reshape(N)
    tagged = _sc_gather()(tag_embedding, ids)
    out = _tc_call(
        word_embedding.reshape(N, H),
        tagged,
        att_gamma.reshape(1, H), att_beta.reshape(1, H),
        W1, b1.reshape(1, INTER),
        W2, b2.reshape(1, H),
        ff_gamma.reshape(1, H), ff_beta.reshape(1, H),
        Wout, bout.reshape(1, NUM_LABELS),
    )
    return out.reshape(B, S, NUM_LABELS)
